# Initial kernel scaffold; baseline (speedup 1.0000x reference)
import jax
import jax.numpy as jnp
from jax.experimental import pallas as pl


def kernel(pred_boxes, pred_logits, gt_boxes, gt_labels, priors_xy):
    raise NotImplementedError("write your pallas kernel here")


# speedup vs baseline: 1.0839x; 1.0839x over previous
import jax
import jax.numpy as jnp
from jax.experimental import pallas as pl

B = 32
N = 20000
C = 81
NEG_POS_RATIO = 3
BOX_LOSS_WEIGHT = 1.0
CENTER_VARIANCE = 0.1
SIZE_VARIANCE = 0.2
EPS = 1e-07

_M = B * N
_R = 2560
_G1 = _M // _R

_INT_MIN = -2147483648

_ATAN_C = (
    0.9999999880821422, -0.33333120777630176, 0.1999371609301848,
    -0.14213195870325923, 0.10681419898649894, -0.07596807640006334,
    0.04385557421611493, -0.01682743386372624, 0.003049964511659606,
)


def _atan_pos(x):
    inv = x > 1.0
    z = jnp.where(inv, 1.0 / x, x)
    z2 = z * z
    p = jnp.float32(_ATAN_C[-1])
    for c in _ATAN_C[-2::-1]:
        p = p * z2 + jnp.float32(c)
    a = p * z
    return jnp.where(inv, jnp.float32(jnp.pi / 2) - a, a)


def _losses_body(x_ref, lab_ref, bg_ref, ce_ref):
    x = x_ref[...]
    m = jnp.max(x, axis=1, keepdims=True)
    e = jnp.exp(x - m)
    lse = m + jnp.log(jnp.sum(e, axis=1, keepdims=True))
    lab = lab_ref[...]
    cidx = jax.lax.broadcasted_iota(jnp.int32, (_R, C), 1)
    xl = jnp.sum(jnp.where(cidx == lab, x, 0.0), axis=1, keepdims=True)
    bg_ref[...] = lse - x[:, 0:1]
    ce_ref[...] = lse - xl


def _monotone_key(f):
    b = jax.lax.bitcast_convert_type(f, jnp.int32)
    return jnp.where(b >= 0, b, jnp.bitwise_xor(jnp.invert(b), jnp.int32(_INT_MIN)))


def _reduce_body(bg_ref, ce_ref, lab_ref, pt_ref, gt_ref, pr_ref,
                 cls_ref, box_ref, np_ref):
    labels = lab_ref[...]
    pos = labels > 0
    npos_row = jnp.sum(pos.astype(jnp.int32), axis=1, keepdims=True)
    k = jnp.minimum(npos_row * NEG_POS_RATIO, N)

    s = _monotone_key(bg_ref[...])
    s = jnp.where(pos, jnp.int32(_INT_MIN), s)

    def bit_step(i, t_u):
        b = 31 - i
        cand_u = t_u | jnp.left_shift(jnp.int32(1), b)
        cand_s = jnp.bitwise_xor(cand_u, jnp.int32(_INT_MIN))
        cnt = jnp.sum((s >= cand_s).astype(jnp.int32), axis=1, keepdims=True)
        return jnp.where(cnt >= k, cand_u, t_u)

    t_u = jax.lax.fori_loop(0, 32, bit_step, jnp.zeros((B, 1), jnp.int32))
    t_s = jnp.bitwise_xor(t_u, jnp.int32(_INT_MIN))
    cnt_gt = jnp.sum((s > t_s).astype(jnp.int32), axis=1, keepdims=True)
    need = k - cnt_gt
    tie = s == t_s
    idx = jax.lax.broadcasted_iota(jnp.int32, (B, N), 1)

    def idx_step(i, t2):
        b = 14 - i
        cand = t2 | jnp.left_shift(jnp.int32(1), b)
        cnt = jnp.sum((tie & (idx >= cand)).astype(jnp.int32), axis=1,
                      keepdims=True)
        return jnp.where(cnt >= need, cand, t2)

    t2 = jax.lax.fori_loop(0, 15, idx_step, jnp.zeros((B, 1), jnp.int32))
    neg = (s > t_s) | (tie & (idx >= t2))
    mask = pos | neg
    cls_ref[...] = jnp.sum(jnp.where(mask, ce_ref[...], 0.0)).reshape(1, 1)

    eps = jnp.float32(EPS)
    pcx = pr_ref[0, :, :]
    pcy = pr_ref[1, :, :]
    pw = pr_ref[2, :, :]
    ph = pr_ref[3, :, :]
    cx = pt_ref[0] * jnp.float32(CENTER_VARIANCE) * pw + pcx
    cy = pt_ref[1] * jnp.float32(CENTER_VARIANCE) * ph + pcy
    w = jnp.exp(pt_ref[2] * jnp.float32(SIZE_VARIANCE)) * pw
    h = jnp.exp(pt_ref[3] * jnp.float32(SIZE_VARIANCE)) * ph
    x1 = cx - w * 0.5
    y1 = cy - h * 0.5
    x2 = cx + w * 0.5
    y2 = cy + h * 0.5
    gx1 = gt_ref[0]
    gy1 = gt_ref[1]
    gx2 = gt_ref[2]
    gy2 = gt_ref[3]

    ow = jnp.clip(jnp.minimum(x2, gx2) - jnp.maximum(x1, gx1), 0.0, None)
    oh = jnp.clip(jnp.minimum(y2, gy2) - jnp.maximum(y1, gy1), 0.0, None)
    overlap = ow * oh
    ap = (x2 - x1) * (y2 - y1)
    ag = (gx2 - gx1) * (gy2 - gy1)
    union = ap + ag - overlap + eps
    ious = overlap / union
    ew = jnp.clip(jnp.maximum(x2, gx2) - jnp.minimum(x1, gx1), 0.0, None)
    eh = jnp.clip(jnp.maximum(y2, gy2) - jnp.minimum(y1, gy1), 0.0, None)
    c2 = ew * ew + eh * eh + eps
    rho2 = ((x1 + x2 - gx1 - gx2) ** 2 + (y1 + y2 - gy1 - gy2) ** 2) * 0.25
    w1 = x2 - x1
    h1 = y2 - y1 + eps
    w2 = gx2 - gx1
    h2 = gy2 - gy1 + eps
    factor = jnp.float32(4.0 / (float(jnp.pi) ** 2))
    v = factor * (_atan_pos(w2 / h2) - _atan_pos(w1 / h1)) ** 2
    alpha = v / (1.0 - ious + v + eps)
    loss = 1.0 - ious + rho2 / c2 + alpha * v
    box_ref[...] = jnp.sum(jnp.where(pos, loss, 0.0)).reshape(1, 1)
    np_ref[...] = jnp.sum(npos_row).astype(jnp.float32).reshape(1, 1)


def kernel(pred_boxes, pred_logits, gt_boxes, gt_labels, priors_xy):
    x2d = pred_logits.reshape(_M, C)
    lab2d = gt_labels.reshape(_M, 1).astype(jnp.int32)

    bg, ce = pl.pallas_call(
        _losses_body,
        grid=(_G1,),
        in_specs=[
            pl.BlockSpec((_R, C), lambda i: (i, 0)),
            pl.BlockSpec((_R, 1), lambda i: (i, 0)),
        ],
        out_specs=[
            pl.BlockSpec((_R, 1), lambda i: (i, 0)),
            pl.BlockSpec((_R, 1), lambda i: (i, 0)),
        ],
        out_shape=[
            jax.ShapeDtypeStruct((_M, 1), jnp.float32),
            jax.ShapeDtypeStruct((_M, 1), jnp.float32),
        ],
    )(x2d, lab2d)

    bg = bg.reshape(B, N)
    ce = ce.reshape(B, N)
    pt = jnp.moveaxis(pred_boxes, 2, 0)
    gt = jnp.moveaxis(gt_boxes, 2, 0)
    pr = jnp.transpose(priors_xy).reshape(4, 1, N)

    cls_s, box_s, np_s = pl.pallas_call(
        _reduce_body,
        in_specs=[
            pl.BlockSpec((B, N), lambda: (0, 0)),
            pl.BlockSpec((B, N), lambda: (0, 0)),
            pl.BlockSpec((B, N), lambda: (0, 0)),
            pl.BlockSpec((4, B, N), lambda: (0, 0, 0)),
            pl.BlockSpec((4, B, N), lambda: (0, 0, 0)),
            pl.BlockSpec((4, 1, N), lambda: (0, 0, 0)),
        ],
        out_specs=[pl.BlockSpec((1, 1), lambda: (0, 0))] * 3,
        out_shape=[jax.ShapeDtypeStruct((1, 1), jnp.float32)] * 3,
    )(bg, ce, gt_labels.astype(jnp.int32), pt, gt, pr)

    num_pos = np_s[0, 0]
    bl = jnp.float32(BOX_LOSS_WEIGHT) * box_s[0, 0] / num_pos
    cl = cls_s[0, 0] / num_pos
    return (bl, cl, bl + cl)
